# baseline (speedup 1.0000x reference)
import functools

import jax
import jax.numpy as jnp
from jax import lax
from jax.experimental import pallas as pl
from jax.experimental.pallas import tpu as pltpu
from jax.experimental.pallas import tpu_sc as plsc

NC = 2
NS = 16
NW = NC * NS
L = 16
CN = 4


def _sc_segment_sums(ne2, et2, ef2, B, NB, D, DT, DE):
  nch = -(-B // (NW * CN))
  npw = nch * CN

  mesh = plsc.VectorSubcoreMesh(
      core_axis_name="c", subcore_axis_name="s",
      num_cores=NC, num_subcores=NS)
  out_type = (
      jax.ShapeDtypeStruct((B, D), jnp.float32),
      jax.ShapeDtypeStruct((B, DT), jnp.float32),
      jax.ShapeDtypeStruct((B, DE), jnp.float32),
  )
  scratch = [
      pltpu.VMEM((CN * NB, D), jnp.float32),
      pltpu.VMEM((CN * NB, DT), jnp.float32),
      pltpu.VMEM((CN * NB, DE), jnp.float32),
      pltpu.VMEM((CN, D), jnp.float32),
      pltpu.VMEM((CN, DT), jnp.float32),
      pltpu.VMEM((CN, DE), jnp.float32),
  ]
  kd = D // L
  kt = DT // L

  @functools.partial(pl.kernel, out_type=out_type, mesh=mesh,
                     scratch_types=scratch)
  def k(ne_h, et_h, ef_h, ns_h, ts_h, es_h, bne, bet, bef, one, ote, oef):
    wid = lax.axis_index("s") * NC + lax.axis_index("c")
    base = wid * npw

    def chunk(ch, carry):
      start = jnp.minimum(base + ch * CN, B - CN)
      r0 = start * NB
      pltpu.sync_copy(ne_h.at[pl.ds(r0, CN * NB), :], bne)
      pltpu.sync_copy(et_h.at[pl.ds(r0, CN * NB), :], bet)
      pltpu.sync_copy(ef_h.at[pl.ds(r0, CN * NB), :], bef)
      for i in range(CN):
        row0 = i * NB

        def nbody(n, accs):
          r = row0 + n
          a = tuple(accs[d] + bne[r, pl.ds(d * L, L)] for d in range(kd))
          b = tuple(accs[kd + d] + bet[r, pl.ds(d * L, L)] for d in range(kt))
          c = (accs[kd + kt] + bef[r, :],)
          return a + b + c

        z = jnp.zeros((L,), jnp.float32)
        accs = lax.fori_loop(0, NB, nbody, (z,) * (kd + kt + 1), unroll=2)
        for d in range(kd):
          one[i, pl.ds(d * L, L)] = accs[d]
        for d in range(kt):
          ote[i, pl.ds(d * L, L)] = accs[kd + d]
        oef[i, :] = accs[kd + kt]
      pltpu.sync_copy(one, ns_h.at[pl.ds(start, CN), :])
      pltpu.sync_copy(ote, ts_h.at[pl.ds(start, CN), :])
      pltpu.sync_copy(oef, es_h.at[pl.ds(start, CN), :])
      return carry

    lax.fori_loop(0, nch, chunk, 0)

  return k(ne2, et2, ef2)


def _tc_head(ns, ts, es, src, tm, W1, b1, W2, b2, B, NB, D, DT, DE, tb):
  def body(ns_r, ts_r, es_r, src_r, tm_r, w1_r, b1_r, w2_r, b2_r, out_r):
    f32 = jnp.float32
    acc = jnp.dot(ns_r[...], w1_r[0:D, :], preferred_element_type=f32)
    acc = acc + jnp.dot(ts_r[...], w1_r[D:D + DT, :],
                        preferred_element_type=f32)
    acc = acc + jnp.dot(es_r[...], w1_r[D + DT:D + DT + DE, :],
                        preferred_element_type=f32)
    h = jnp.maximum(acc + f32(NB) * b1_r[0, :][None, :], 0.0)
    o = jnp.dot(h, w2_r[0:D, :], preferred_element_type=f32)
    o = o + jnp.dot(src_r[...], w2_r[D:2 * D, :], preferred_element_type=f32)
    o = o + jnp.dot(tm_r[...], w2_r[2 * D:2 * D + DT, :],
                    preferred_element_type=f32)
    out_r[...] = o + b2_r[0, :][None, :]

  return pl.pallas_call(
      body,
      grid=(B // tb,),
      in_specs=[
          pl.BlockSpec((tb, D), lambda i: (i, 0)),
          pl.BlockSpec((tb, DT), lambda i: (i, 0)),
          pl.BlockSpec((tb, DE), lambda i: (i, 0)),
          pl.BlockSpec((tb, D), lambda i: (i, 0)),
          pl.BlockSpec((tb, DT), lambda i: (i, 0)),
          pl.BlockSpec((D + DT + DE, D), lambda i: (0, 0)),
          pl.BlockSpec((1, D), lambda i: (0, 0)),
          pl.BlockSpec((2 * D + DT, D), lambda i: (0, 0)),
          pl.BlockSpec((1, D), lambda i: (0, 0)),
      ],
      out_specs=pl.BlockSpec((tb, D), lambda i: (i, 0)),
      out_shape=jax.ShapeDtypeStruct((B, D), jnp.float32),
  )(ns, ts, es, src, tm, W1, b1, W2, b2)


def kernel(n_layer, source_node_features, source_nodes_time_embedding,
           neighbor_embeddings, edge_time_embeddings, edge_features, mask,
           W1, b1, W2, b2):
  B, NB, D = neighbor_embeddings.shape
  DT = edge_time_embeddings.shape[2]
  DE = edge_features.shape[2]

  ne2 = neighbor_embeddings.reshape(B * NB, D)
  et2 = edge_time_embeddings.reshape(B * NB, DT)
  ef2 = edge_features.reshape(B * NB, DE)
  ns, ts, es = _sc_segment_sums(ne2, et2, ef2, B, NB, D, DT, DE)

  src = source_node_features
  tm = jnp.squeeze(source_nodes_time_embedding, axis=1)
  tb = 1000
  return _tc_head(ns, ts, es, src, tm, W1, b1.reshape(1, D), W2,
                  b2.reshape(1, D), B, NB, D, DT, DE, tb)


# speedup vs baseline: 1.6536x; 1.6536x over previous
import functools

import jax
import jax.numpy as jnp
from jax import lax
from jax.experimental import pallas as pl
from jax.experimental.pallas import tpu as pltpu
from jax.experimental.pallas import tpu_sc as plsc

NC = 2
NS = 16
NW = NC * NS
L = 16
CN = 4


def _sc_segment_sums(ne2, et2, ef2, B, NB, D, DT, DE):
  nch = -(-B // (NW * CN))
  nch = nch + (nch % 2)
  npw = nch * CN

  assert nch % 2 == 0

  mesh = plsc.VectorSubcoreMesh(
      core_axis_name="c", subcore_axis_name="s",
      num_cores=NC, num_subcores=NS)
  out_type = (
      jax.ShapeDtypeStruct((B, D), jnp.float32),
      jax.ShapeDtypeStruct((B, DT), jnp.float32),
      jax.ShapeDtypeStruct((B, DE), jnp.float32),
  )
  scratch = [
      [pltpu.VMEM((CN * NB, D), jnp.float32)] * 2,
      [pltpu.VMEM((CN * NB, DT), jnp.float32)] * 2,
      [pltpu.VMEM((CN * NB, DE), jnp.float32)] * 2,
      [pltpu.VMEM((CN, D), jnp.float32)] * 2,
      [pltpu.VMEM((CN, DT), jnp.float32)] * 2,
      [pltpu.VMEM((CN, DE), jnp.float32)] * 2,
      [pltpu.SemaphoreType.DMA] * 2,
      [pltpu.SemaphoreType.DMA] * 2,
  ]
  kd = D // L
  kt = DT // L

  @functools.partial(pl.kernel, out_type=out_type, mesh=mesh,
                     scratch_types=scratch)
  def k(ne_h, et_h, ef_h, ns_h, ts_h, es_h, bne, bet, bef, one, ote, oef,
        sin, sout):
    wid = lax.axis_index("s") * NC + lax.axis_index("c")
    base = wid * npw

    def in_copies(ch, b):
      start = jnp.minimum(base + ch * CN, B - CN)
      r0 = start * NB
      return (
          pltpu.make_async_copy(ne_h.at[pl.ds(r0, CN * NB), :], bne[b],
                                sin[b]),
          pltpu.make_async_copy(et_h.at[pl.ds(r0, CN * NB), :], bet[b],
                                sin[b]),
          pltpu.make_async_copy(ef_h.at[pl.ds(r0, CN * NB), :], bef[b],
                                sin[b]),
      )

    def out_copies(ch, b):
      start = jnp.minimum(base + ch * CN, B - CN)
      return (
          pltpu.make_async_copy(one[b], ns_h.at[pl.ds(start, CN), :],
                                sout[b]),
          pltpu.make_async_copy(ote[b], ts_h.at[pl.ds(start, CN), :],
                                sout[b]),
          pltpu.make_async_copy(oef[b], es_h.at[pl.ds(start, CN), :],
                                sout[b]),
      )

    def compute(b):
      for i in range(CN):
        row0 = i * NB

        def nbody(n, accs):
          r = row0 + n
          aa = tuple(
              accs[d] + bne[b][r, pl.ds(d * L, L)] for d in range(kd))
          bb = tuple(
              accs[kd + d] + bet[b][r, pl.ds(d * L, L)] for d in range(kt))
          cc = (accs[kd + kt] + bef[b][r, :],)
          return aa + bb + cc

        z = jnp.zeros((L,), jnp.float32)
        accs = lax.fori_loop(0, NB, nbody, (z,) * (kd + kt + 1), unroll=2)
        for d in range(kd):
          one[b][i, pl.ds(d * L, L)] = accs[d]
        for d in range(kt):
          ote[b][i, pl.ds(d * L, L)] = accs[kd + d]
        oef[b][i, :] = accs[kd + kt]

    def slot(ch, b, t):
      for c in in_copies(ch, b):
        c.wait()

      @pl.when(t > 0)
      def _():
        for c in out_copies(ch - 2, b):
          c.wait()

      compute(b)
      for c in out_copies(ch, b):
        c.start()

      @pl.when(ch + 2 < nch)
      def _():
        for c in in_copies(ch + 2, b):
          c.start()

    for c in in_copies(0, 0):
      c.start()
    for c in in_copies(1, 1):
      c.start()

    def pair(t, carry):
      slot(2 * t, 0, t)
      slot(2 * t + 1, 1, t)
      return carry

    lax.fori_loop(0, nch // 2, pair, 0)
    for c in out_copies(nch - 2, 0):
      c.wait()
    for c in out_copies(nch - 1, 1):
      c.wait()

  return k(ne2, et2, ef2)


def _tc_head(ns, ts, es, src, tm, W1, b1, W2, b2, B, NB, D, DT, DE, tb):
  def body(ns_r, ts_r, es_r, src_r, tm_r, w1_r, b1_r, w2_r, b2_r, out_r):
    f32 = jnp.float32
    acc = jnp.dot(ns_r[...], w1_r[0:D, :], preferred_element_type=f32)
    acc = acc + jnp.dot(ts_r[...], w1_r[D:D + DT, :],
                        preferred_element_type=f32)
    acc = acc + jnp.dot(es_r[...], w1_r[D + DT:D + DT + DE, :],
                        preferred_element_type=f32)
    h = jnp.maximum(acc + f32(NB) * b1_r[0, :][None, :], 0.0)
    o = jnp.dot(h, w2_r[0:D, :], preferred_element_type=f32)
    o = o + jnp.dot(src_r[...], w2_r[D:2 * D, :], preferred_element_type=f32)
    o = o + jnp.dot(tm_r[...], w2_r[2 * D:2 * D + DT, :],
                    preferred_element_type=f32)
    out_r[...] = o + b2_r[0, :][None, :]

  return pl.pallas_call(
      body,
      grid=(B // tb,),
      in_specs=[
          pl.BlockSpec((tb, D), lambda i: (i, 0)),
          pl.BlockSpec((tb, DT), lambda i: (i, 0)),
          pl.BlockSpec((tb, DE), lambda i: (i, 0)),
          pl.BlockSpec((tb, D), lambda i: (i, 0)),
          pl.BlockSpec((tb, DT), lambda i: (i, 0)),
          pl.BlockSpec((D + DT + DE, D), lambda i: (0, 0)),
          pl.BlockSpec((1, D), lambda i: (0, 0)),
          pl.BlockSpec((2 * D + DT, D), lambda i: (0, 0)),
          pl.BlockSpec((1, D), lambda i: (0, 0)),
      ],
      out_specs=pl.BlockSpec((tb, D), lambda i: (i, 0)),
      out_shape=jax.ShapeDtypeStruct((B, D), jnp.float32),
  )(ns, ts, es, src, tm, W1, b1, W2, b2)


def kernel(n_layer, source_node_features, source_nodes_time_embedding,
           neighbor_embeddings, edge_time_embeddings, edge_features, mask,
           W1, b1, W2, b2):
  B, NB, D = neighbor_embeddings.shape
  DT = edge_time_embeddings.shape[2]
  DE = edge_features.shape[2]

  ne2 = neighbor_embeddings.reshape(B * NB, D)
  et2 = edge_time_embeddings.reshape(B * NB, DT)
  ef2 = edge_features.reshape(B * NB, DE)
  ns, ts, es = _sc_segment_sums(ne2, et2, ef2, B, NB, D, DT, DE)

  src = source_node_features
  tm = jnp.squeeze(source_nodes_time_embedding, axis=1)
  tb = 1000
  return _tc_head(ns, ts, es, src, tm, W1, b1.reshape(1, D), W2,
                  b2.reshape(1, D), B, NB, D, DT, DE, tb)


# speedup vs baseline: 2.1763x; 1.3161x over previous
import functools

import jax
import jax.numpy as jnp
from jax import lax
from jax.experimental import pallas as pl
from jax.experimental.pallas import tpu as pltpu
from jax.experimental.pallas import tpu_sc as plsc

NC = 2
NS = 16
NW = NC * NS
L = 16
CN = 8


def _sc_neighbor_sum(ne2, B, NB, D):
  nch = -(-B // (NW * CN))
  nch = nch + (nch % 2)
  npw = nch * CN

  mesh = plsc.VectorSubcoreMesh(
      core_axis_name="c", subcore_axis_name="s",
      num_cores=NC, num_subcores=NS)
  scratch = [
      [pltpu.VMEM((CN * NB, D), jnp.float32)] * 2,
      [pltpu.VMEM((CN, D), jnp.float32)] * 2,
      [pltpu.SemaphoreType.DMA] * 2,
      [pltpu.SemaphoreType.DMA] * 2,
  ]
  kd = D // L

  @functools.partial(
      pl.kernel,
      out_type=jax.ShapeDtypeStruct((B, D), jnp.float32),
      mesh=mesh, scratch_types=scratch)
  def k(ne_h, ns_h, bne, one, sin, sout):
    wid = lax.axis_index("s") * NC + lax.axis_index("c")
    base = wid * npw

    def in_copy(ch, b):
      start = jnp.minimum(base + ch * CN, B - CN)
      return pltpu.make_async_copy(
          ne_h.at[pl.ds(start * NB, CN * NB), :], bne[b], sin[b])

    def out_copy(ch, b):
      start = jnp.minimum(base + ch * CN, B - CN)
      return pltpu.make_async_copy(
          one[b], ns_h.at[pl.ds(start, CN), :], sout[b])

    def compute(b):
      for i in range(CN):
        row0 = i * NB

        def nbody(n, accs):
          r = row0 + n
          return tuple(
              accs[d] + bne[b][r, pl.ds(d * L, L)] for d in range(kd))

        z = jnp.zeros((L,), jnp.float32)
        accs = lax.fori_loop(0, NB, nbody, (z,) * kd, unroll=4)
        for d in range(kd):
          one[b][i, pl.ds(d * L, L)] = accs[d]

    def slot(ch, b, t):
      in_copy(ch, b).wait()

      @pl.when(t > 0)
      def _():
        out_copy(ch - 2, b).wait()

      compute(b)
      out_copy(ch, b).start()

      @pl.when(ch + 2 < nch)
      def _():
        in_copy(ch + 2, b).start()

    in_copy(0, 0).start()
    in_copy(1, 1).start()

    def pair(t, carry):
      slot(2 * t, 0, t)
      slot(2 * t + 1, 1, t)
      return carry

    lax.fori_loop(0, nch // 2, pair, 0)
    out_copy(nch - 2, 0).wait()
    out_copy(nch - 1, 1).wait()

  return k(ne2)


def _tc_edge_partial(et2, ef2, W1bc, b1, B, NB, DT, DE, D, tb):
  def body(et_r, ef_r, w_r, b1_r, out_r):
    f32 = jnp.float32
    ts = jnp.sum(et_r[...].reshape(tb, NB, DT), axis=1)
    es = jnp.sum(ef_r[...].reshape(tb, NB, DE), axis=1)
    acc = jnp.dot(ts, w_r[0:DT, :], preferred_element_type=f32)
    acc = acc + jnp.dot(es, w_r[DT:DT + DE, :], preferred_element_type=f32)
    out_r[...] = acc + f32(NB) * b1_r[0, :][None, :]

  return pl.pallas_call(
      body,
      grid=(B // tb,),
      in_specs=[
          pl.BlockSpec((tb * NB, DT), lambda i: (i, 0)),
          pl.BlockSpec((tb * NB, DE), lambda i: (i, 0)),
          pl.BlockSpec((DT + DE, D), lambda i: (0, 0)),
          pl.BlockSpec((1, D), lambda i: (0, 0)),
      ],
      out_specs=pl.BlockSpec((tb, D), lambda i: (i, 0)),
      out_shape=jax.ShapeDtypeStruct((B, D), jnp.float32),
  )(et2, ef2, W1bc, b1)


def _tc_head(ns, P, src, tm, W1a, W2, b2, B, D, DT, tb):
  def body(ns_r, p_r, src_r, tm_r, w1_r, w2_r, b2_r, out_r):
    f32 = jnp.float32
    h = jnp.maximum(
        jnp.dot(ns_r[...], w1_r[...], preferred_element_type=f32) + p_r[...],
        0.0)
    o = jnp.dot(h, w2_r[0:D, :], preferred_element_type=f32)
    o = o + jnp.dot(src_r[...], w2_r[D:2 * D, :], preferred_element_type=f32)
    o = o + jnp.dot(tm_r[...], w2_r[2 * D:2 * D + DT, :],
                    preferred_element_type=f32)
    out_r[...] = o + b2_r[0, :][None, :]

  return pl.pallas_call(
      body,
      grid=(B // tb,),
      in_specs=[
          pl.BlockSpec((tb, D), lambda i: (i, 0)),
          pl.BlockSpec((tb, D), lambda i: (i, 0)),
          pl.BlockSpec((tb, D), lambda i: (i, 0)),
          pl.BlockSpec((tb, DT), lambda i: (i, 0)),
          pl.BlockSpec((D, D), lambda i: (0, 0)),
          pl.BlockSpec((2 * D + DT, D), lambda i: (0, 0)),
          pl.BlockSpec((1, D), lambda i: (0, 0)),
      ],
      out_specs=pl.BlockSpec((tb, D), lambda i: (i, 0)),
      out_shape=jax.ShapeDtypeStruct((B, D), jnp.float32),
  )(ns, P, src, tm, W1a, W2, b2)


def kernel(n_layer, source_node_features, source_nodes_time_embedding,
           neighbor_embeddings, edge_time_embeddings, edge_features, mask,
           W1, b1, W2, b2):
  B, NB, D = neighbor_embeddings.shape
  DT = edge_time_embeddings.shape[2]
  DE = edge_features.shape[2]

  ne2 = neighbor_embeddings.reshape(B * NB, D)
  et2 = edge_time_embeddings.reshape(B * NB, DT)
  ef2 = edge_features.reshape(B * NB, DE)

  ns = _sc_neighbor_sum(ne2, B, NB, D)
  P = _tc_edge_partial(et2, ef2, W1[D:, :], b1.reshape(1, D),
                       B, NB, DT, DE, D, tb=200)

  src = source_node_features
  tm = jnp.squeeze(source_nodes_time_embedding, axis=1)
  return _tc_head(ns, P, src, tm, W1[0:D, :], W2, b2.reshape(1, D),
                  B, D, DT, tb=1000)
